# baseline (speedup 1.0000x reference)
import dataclasses
import functools

import jax
import jax.numpy as jnp
from jax import lax
from jax.experimental import pallas as pl
from jax.experimental.pallas import tpu as pltpu
from jax.experimental.pallas import tpu_sc as plsc

N = 10000
NFEAT = 128
NHID = 128
NCLASS = 64
E = 320000

NC = 2
NS = 16
LANES = 16

CHUNK = 128
SUP = 8
E_PAD = 327680
TOT_SUP = E_PAD // (SUP * CHUNK)
SUP_C0 = 18
SUP_C1 = TOT_SUP // NS - SUP_C0

N_PAD = 10240
ROWS_PER_SUB = N_PAD // NS
ZCHUNK = 128

NBUF = 2


def _make_spmm(D):
    mesh = plsc.VectorSubcoreMesh(core_axis_name="c", subcore_axis_name="s")

    cp = pltpu.CompilerParams()
    if "needs_layout_passes" in pltpu.CompilerParams.__dataclass_fields__:
        cp = dataclasses.replace(cp, needs_layout_passes=False)

    @functools.partial(
        pl.kernel,
        compiler_params=cp,
        out_type=jax.ShapeDtypeStruct((NC, N_PAD, D), jnp.float32),
        mesh=mesh,
        scratch_types=[
            pltpu.VMEM((SUP, CHUNK), jnp.int32),
            pltpu.VMEM((SUP, CHUNK), jnp.int32),
            pltpu.VMEM((2 * SUP, CHUNK // 2), jnp.int32),
            pltpu.VMEM((2 * SUP, CHUNK // 2), jnp.int32),
            pltpu.VMEM((SUP * CHUNK,), jnp.float32),
            pltpu.VMEM((SUP * CHUNK,), jnp.float32),
            pltpu.VMEM((CHUNK, D), jnp.float32),
            pltpu.VMEM((CHUNK, D), jnp.float32),
            pltpu.VMEM_SHARED((N_PAD, D), jnp.float32),
            pltpu.SemaphoreType.DMA,
            pltpu.SemaphoreType.DMA,
            pltpu.SemaphoreType.DMA,
            pltpu.SemaphoreType.DMA,
            pltpu.SemaphoreType.DMA,
            pltpu.SemaphoreType.DMA,
        ],
    )
    def spmm(h_hbm, src_hbm, dst_hbm, w_hbm, out_hbm,
             src0, src1, dst0, dst1, w0, w1, r0, r1,
             acc_sh, sem_i0, sem_i1, g0, g1, s0, s1):
        cid = lax.axis_index("c")
        sid = lax.axis_index("s")
        nsup = jnp.where(cid == 0, SUP_C0, SUP_C1)
        base = jnp.where(cid == 0, sid * SUP_C0, NS * SUP_C0 + sid * SUP_C1)
        rows = (r0, r1)
        gsem = (g0, g1)
        ssem = (s0, s1)

        zeros16 = jnp.zeros((LANES,), jnp.float32)

        @pl.loop(0, ZCHUNK)
        def _(r):
            for c in range(D // LANES):
                r0.at[r, pl.ds(c * LANES, LANES)][...] = zeros16

        for k in range(ROWS_PER_SUB // ZCHUNK):
            pltpu.sync_copy(
                r0, acc_sh.at[pl.ds(sid * ROWS_PER_SUB + k * ZCHUNK, ZCHUNK)])
        plsc.subcore_barrier()

        def fetch_idx(sup, sb, db, wb, sem):
            g = (base + sup) * SUP
            pltpu.async_copy(src_hbm.at[pl.ds(g, SUP)], sb, sem)
            pltpu.async_copy(dst_hbm.at[pl.ds(2 * g, 2 * SUP)], db, sem)
            pltpu.async_copy(w_hbm.at[pl.ds(g * CHUNK, SUP * CHUNK)], wb, sem)

        def wait_idx(sb, db, wb, sem):
            pltpu.make_async_copy(src_hbm.at[pl.ds(0, SUP)], sb, sem).wait()
            pltpu.make_async_copy(dst_hbm.at[pl.ds(0, 2 * SUP)], db, sem).wait()
            pltpu.make_async_copy(
                w_hbm.at[pl.ds(0, SUP * CHUNK)], wb, sem).wait()

        def do_super(sb, db, wb):
            for j in range(NBUF - 1):
                pltpu.async_copy(h_hbm.at[sb.at[j]], rows[j], gsem[j])
            HC = CHUNK // 2
            for j in range(SUP):
                p = j % NBUF
                rb = rows[p]
                pltpu.make_async_copy(h_hbm.at[sb.at[0]], rb, gsem[p]).wait()
                if j + NBUF - 1 < SUP:
                    q = (j + NBUF - 1) % NBUF
                    if j >= 1:
                        pltpu.make_async_copy(
                            rows[q].at[pl.ds(0, HC)],
                            acc_sh.at[db.at[0]], ssem[q]).wait()
                    pltpu.async_copy(h_hbm.at[sb.at[j + NBUF - 1]],
                                     rows[q], gsem[q])

                @plsc.parallel_loop(0, HC, unroll=4)
                def _(e):
                    widx = jnp.full((LANES,), j * CHUNK + e, dtype=jnp.int32)
                    wv = plsc.load_gather(wb, [widx])
                    for c in range(D // LANES):
                        sl = pl.ds(c * LANES, LANES)
                        rb.at[e, sl][...] = rb.at[e, sl][...] * wv

                pltpu.async_copy(rb.at[pl.ds(0, HC)],
                                 acc_sh.at[db.at[2 * j]], ssem[p], add=True)

                @plsc.parallel_loop(HC, CHUNK, unroll=4)
                def _(e):
                    widx = jnp.full((LANES,), j * CHUNK + e, dtype=jnp.int32)
                    wv = plsc.load_gather(wb, [widx])
                    for c in range(D // LANES):
                        sl = pl.ds(c * LANES, LANES)
                        rb.at[e, sl][...] = rb.at[e, sl][...] * wv

                pltpu.sync_copy(rb.at[pl.ds(HC, HC)],
                                acc_sh.at[db.at[2 * j + 1]], add=True)
            for p2 in range(NBUF):
                pltpu.make_async_copy(rows[p2].at[pl.ds(0, CHUNK // 2)],
                                      acc_sh.at[db.at[0]], ssem[p2]).wait()

        fetch_idx(0, src0, dst0, w0, sem_i0)
        nsup_even = (nsup // 2) * 2

        @pl.loop(0, nsup_even, step=2)
        def _(sup):
            wait_idx(src0, dst0, w0, sem_i0)
            fetch_idx(sup + 1, src1, dst1, w1, sem_i1)
            do_super(src0, dst0, w0)
            wait_idx(src1, dst1, w1, sem_i1)

            @pl.when(sup + 2 < nsup)
            def _():
                fetch_idx(sup + 2, src0, dst0, w0, sem_i0)

            do_super(src1, dst1, w1)

        @pl.when(nsup_even < nsup)
        def _():
            wait_idx(src0, dst0, w0, sem_i0)
            do_super(src0, dst0, w0)

        plsc.subcore_barrier()
        for k in range(ROWS_PER_SUB // ZCHUNK):
            row0 = sid * ROWS_PER_SUB + k * ZCHUNK
            pltpu.sync_copy(acc_sh.at[pl.ds(row0, ZCHUNK)],
                            out_hbm.at[cid, pl.ds(row0, ZCHUNK)])

    return spmm


_spmm = _make_spmm(NHID)

_ROWS = 1000


def _mm1(x, W1):
    def body(x_ref, w_ref, o_ref):
        o_ref[...] = jnp.dot(x_ref[...], w_ref[...],
                             preferred_element_type=jnp.float32)

    return pl.pallas_call(
        body,
        grid=(N // _ROWS,),
        in_specs=[pl.BlockSpec((_ROWS, NFEAT), lambda i: (i, 0)),
                  pl.BlockSpec((NFEAT, NHID), lambda i: (0, 0))],
        out_specs=pl.BlockSpec((_ROWS, NHID), lambda i: (i, 0)),
        out_shape=jax.ShapeDtypeStruct((N, NHID), jnp.float32),
    )(x, W1)


def _mid(p, b1, W2):
    def body(p_ref, b_ref, w_ref, o_ref):
        h = p_ref[0] + p_ref[1] + b_ref[...]
        h = jnp.maximum(h, 0.0)
        o_ref[...] = jnp.dot(h, w_ref[...], preferred_element_type=jnp.float32)

    return pl.pallas_call(
        body,
        grid=(N_PAD // _ROWS,),
        in_specs=[pl.BlockSpec((NC, _ROWS, NHID), lambda i: (0, i, 0)),
                  pl.BlockSpec((1, NHID), lambda i: (0, 0)),
                  pl.BlockSpec((NHID, NHID), lambda i: (0, 0))],
        out_specs=pl.BlockSpec((_ROWS, NHID), lambda i: (i, 0)),
        out_shape=jax.ShapeDtypeStruct((N_PAD, NHID), jnp.float32),
    )(p, b1, W2)


def _logsoftmax_out(q, b2):
    def body(q_ref, b_ref, o_ref):
        z = q_ref[0, :, :NCLASS] + q_ref[1, :, :NCLASS] + b_ref[...]
        m = jnp.max(z, axis=1, keepdims=True)
        s = jnp.sum(jnp.exp(z - m), axis=1, keepdims=True)
        o_ref[...] = z - m - jnp.log(s)

    return pl.pallas_call(
        body,
        grid=(N // _ROWS,),
        in_specs=[pl.BlockSpec((NC, _ROWS, NHID), lambda i: (0, i, 0)),
                  pl.BlockSpec((1, NCLASS), lambda i: (0, 0))],
        out_specs=pl.BlockSpec((_ROWS, NCLASS), lambda i: (i, 0)),
        out_shape=jax.ShapeDtypeStruct((N, NCLASS), jnp.float32),
    )(q, b2)


def kernel(x, edge_index, edge_weight, W1, b1, W2, b2):
    dst = edge_index[0].astype(jnp.int32)
    src = edge_index[1].astype(jnp.int32)
    w = edge_weight.astype(jnp.float32)

    pad = E_PAD - E
    srcp = jnp.concatenate([src, jnp.zeros((pad,), jnp.int32)])
    dstp = jnp.concatenate([dst, jnp.zeros((pad,), jnp.int32)])
    wp = jnp.concatenate([w, jnp.zeros((pad,), jnp.float32)])
    srcp = srcp.reshape(E_PAD // CHUNK, CHUNK)
    dstp = dstp.reshape(E_PAD // (CHUNK // 2), CHUNK // 2)

    W2p = jnp.pad(W2, ((0, 0), (0, NHID - NCLASS)))

    support1 = _mm1(x, W1)
    p = _spmm(support1, srcp, dstp, wp)
    h2 = _mid(p, b1.reshape(1, NHID), W2p)
    q = _spmm(h2, srcp, dstp, wp)
    return _logsoftmax_out(q, b2.reshape(1, NCLASS))


# speedup vs baseline: 1.0390x; 1.0390x over previous
import dataclasses
import functools

import jax
import jax.numpy as jnp
from jax import lax
from jax.experimental import pallas as pl
from jax.experimental.pallas import tpu as pltpu
from jax.experimental.pallas import tpu_sc as plsc

N = 10000
NFEAT = 128
NHID = 128
NCLASS = 64
E = 320000

NC = 2
NS = 16
LANES = 16

CHUNK = 128
SUP = 16
E_PAD = 327680
TOT_SUP = E_PAD // (SUP * CHUNK)
SUP_C0 = 9
SUP_C1 = TOT_SUP // NS - SUP_C0

N_PAD = 10240
ROWS_PER_SUB = N_PAD // NS
ZCHUNK = 128

NBUF = 2


def _make_spmm(D):
    mesh = plsc.VectorSubcoreMesh(core_axis_name="c", subcore_axis_name="s")

    cp = pltpu.CompilerParams()
    if "needs_layout_passes" in pltpu.CompilerParams.__dataclass_fields__:
        cp = dataclasses.replace(cp, needs_layout_passes=False)

    @functools.partial(
        pl.kernel,
        compiler_params=cp,
        out_type=jax.ShapeDtypeStruct((NC, N_PAD, D), jnp.float32),
        mesh=mesh,
        scratch_types=[
            pltpu.VMEM((SUP, CHUNK), jnp.int32),
            pltpu.VMEM((SUP, CHUNK), jnp.int32),
            pltpu.VMEM((SUP, CHUNK), jnp.int32),
            pltpu.VMEM((SUP, CHUNK), jnp.int32),
            pltpu.VMEM((SUP * CHUNK,), jnp.float32),
            pltpu.VMEM((SUP * CHUNK,), jnp.float32),
            pltpu.VMEM((CHUNK, D), jnp.float32),
            pltpu.VMEM((CHUNK, D), jnp.float32),
            pltpu.VMEM_SHARED((N_PAD, D), jnp.float32),
            pltpu.SemaphoreType.DMA,
            pltpu.SemaphoreType.DMA,
            pltpu.SemaphoreType.DMA,
            pltpu.SemaphoreType.DMA,
        ],
    )
    def spmm(h_hbm, src_hbm, dst_hbm, w_hbm, out_hbm,
             src0, src1, dst0, dst1, w0, w1, r0, r1,
             acc_sh, sem_i0, sem_i1, g0, g1):
        cid = lax.axis_index("c")
        sid = lax.axis_index("s")
        nsup = jnp.where(cid == 0, SUP_C0, SUP_C1)
        base = jnp.where(cid == 0, sid * SUP_C0, NS * SUP_C0 + sid * SUP_C1)
        rows = (r0, r1)
        gsem = (g0, g1)

        zeros16 = jnp.zeros((LANES,), jnp.float32)

        @pl.loop(0, ZCHUNK)
        def _(r):
            for c in range(D // LANES):
                r0.at[r, pl.ds(c * LANES, LANES)][...] = zeros16

        for k in range(ROWS_PER_SUB // ZCHUNK):
            pltpu.sync_copy(
                r0, acc_sh.at[pl.ds(sid * ROWS_PER_SUB + k * ZCHUNK, ZCHUNK)])
        plsc.subcore_barrier()

        def fetch_idx(sup, sb, db, wb, sem):
            g = (base + sup) * SUP
            pltpu.async_copy(src_hbm.at[pl.ds(g, SUP)], sb, sem)
            pltpu.async_copy(dst_hbm.at[pl.ds(g, SUP)], db, sem)
            pltpu.async_copy(w_hbm.at[pl.ds(g * CHUNK, SUP * CHUNK)], wb, sem)

        def wait_idx(sb, db, wb, sem):
            pltpu.make_async_copy(src_hbm.at[pl.ds(0, SUP)], sb, sem).wait()
            pltpu.make_async_copy(dst_hbm.at[pl.ds(0, SUP)], db, sem).wait()
            pltpu.make_async_copy(
                w_hbm.at[pl.ds(0, SUP * CHUNK)], wb, sem).wait()

        def do_super(sb, db, wb):
            for j in range(NBUF - 1):
                pltpu.async_copy(h_hbm.at[sb.at[j]], rows[j], gsem[j])
            for j in range(SUP):
                p = j % NBUF
                rb = rows[p]
                pltpu.make_async_copy(h_hbm.at[sb.at[0]], rb, gsem[p]).wait()
                if j + NBUF - 1 < SUP:
                    q = (j + NBUF - 1) % NBUF
                    pltpu.async_copy(h_hbm.at[sb.at[j + NBUF - 1]],
                                     rows[q], gsem[q])

                @plsc.parallel_loop(0, CHUNK, unroll=4)
                def _(e):
                    widx = jnp.full((LANES,), j * CHUNK + e, dtype=jnp.int32)
                    wv = plsc.load_gather(wb, [widx])
                    for c in range(D // LANES):
                        sl = pl.ds(c * LANES, LANES)
                        rb.at[e, sl][...] = rb.at[e, sl][...] * wv

                pltpu.sync_copy(rb, acc_sh.at[db.at[j]], add=True)

        fetch_idx(0, src0, dst0, w0, sem_i0)
        nsup_even = (nsup // 2) * 2

        @pl.loop(0, nsup_even, step=2)
        def _(sup):
            wait_idx(src0, dst0, w0, sem_i0)
            fetch_idx(sup + 1, src1, dst1, w1, sem_i1)
            do_super(src0, dst0, w0)
            wait_idx(src1, dst1, w1, sem_i1)

            @pl.when(sup + 2 < nsup)
            def _():
                fetch_idx(sup + 2, src0, dst0, w0, sem_i0)

            do_super(src1, dst1, w1)

        @pl.when(nsup_even < nsup)
        def _():
            wait_idx(src0, dst0, w0, sem_i0)
            do_super(src0, dst0, w0)

        plsc.subcore_barrier()
        for k in range(ROWS_PER_SUB // ZCHUNK):
            row0 = sid * ROWS_PER_SUB + k * ZCHUNK
            pltpu.sync_copy(acc_sh.at[pl.ds(row0, ZCHUNK)],
                            out_hbm.at[cid, pl.ds(row0, ZCHUNK)])

    return spmm


_spmm = _make_spmm(NHID)

_ROWS = 1000


def _mm1(x, W1):
    def body(x_ref, w_ref, o_ref):
        o_ref[...] = jnp.dot(x_ref[...], w_ref[...],
                             preferred_element_type=jnp.float32)

    return pl.pallas_call(
        body,
        grid=(N // _ROWS,),
        in_specs=[pl.BlockSpec((_ROWS, NFEAT), lambda i: (i, 0)),
                  pl.BlockSpec((NFEAT, NHID), lambda i: (0, 0))],
        out_specs=pl.BlockSpec((_ROWS, NHID), lambda i: (i, 0)),
        out_shape=jax.ShapeDtypeStruct((N, NHID), jnp.float32),
    )(x, W1)


def _mid(p, b1, W2):
    def body(p_ref, b_ref, w_ref, o_ref):
        h = p_ref[0] + p_ref[1] + b_ref[...]
        h = jnp.maximum(h, 0.0)
        o_ref[...] = jnp.dot(h, w_ref[...], preferred_element_type=jnp.float32)

    return pl.pallas_call(
        body,
        grid=(N_PAD // _ROWS,),
        in_specs=[pl.BlockSpec((NC, _ROWS, NHID), lambda i: (0, i, 0)),
                  pl.BlockSpec((1, NHID), lambda i: (0, 0)),
                  pl.BlockSpec((NHID, NHID), lambda i: (0, 0))],
        out_specs=pl.BlockSpec((_ROWS, NHID), lambda i: (i, 0)),
        out_shape=jax.ShapeDtypeStruct((N_PAD, NHID), jnp.float32),
    )(p, b1, W2)


def _logsoftmax_out(q, b2):
    def body(q_ref, b_ref, o_ref):
        z = q_ref[0, :, :NCLASS] + q_ref[1, :, :NCLASS] + b_ref[...]
        m = jnp.max(z, axis=1, keepdims=True)
        s = jnp.sum(jnp.exp(z - m), axis=1, keepdims=True)
        o_ref[...] = z - m - jnp.log(s)

    return pl.pallas_call(
        body,
        grid=(N // _ROWS,),
        in_specs=[pl.BlockSpec((NC, _ROWS, NHID), lambda i: (0, i, 0)),
                  pl.BlockSpec((1, NCLASS), lambda i: (0, 0))],
        out_specs=pl.BlockSpec((_ROWS, NCLASS), lambda i: (i, 0)),
        out_shape=jax.ShapeDtypeStruct((N, NCLASS), jnp.float32),
    )(q, b2)


def kernel(x, edge_index, edge_weight, W1, b1, W2, b2):
    dst = edge_index[0].astype(jnp.int32)
    src = edge_index[1].astype(jnp.int32)
    w = edge_weight.astype(jnp.float32)

    pad = E_PAD - E
    srcp = jnp.concatenate([src, jnp.zeros((pad,), jnp.int32)])
    dstp = jnp.concatenate([dst, jnp.zeros((pad,), jnp.int32)])
    wp = jnp.concatenate([w, jnp.zeros((pad,), jnp.float32)])
    srcp = srcp.reshape(E_PAD // CHUNK, CHUNK)
    dstp = dstp.reshape(E_PAD // CHUNK, CHUNK)

    W2p = jnp.pad(W2, ((0, 0), (0, NHID - NCLASS)))

    support1 = _mm1(x, W1)
    p = _spmm(support1, srcp, dstp, wp)
    h2 = _mid(p, b1.reshape(1, NHID), W2p)
    q = _spmm(h2, srcp, dstp, wp)
    return _logsoftmax_out(q, b2.reshape(1, NCLASS))


# speedup vs baseline: 1.0461x; 1.0069x over previous
import dataclasses
import functools

import jax
import jax.numpy as jnp
from jax import lax
from jax.experimental import pallas as pl
from jax.experimental.pallas import tpu as pltpu
from jax.experimental.pallas import tpu_sc as plsc

N = 10000
NFEAT = 128
NHID = 128
NCLASS = 64
E = 320000

NC = 2
NS = 16
LANES = 16

CHUNK = 128
SUP = 8
E_PAD = 327680
TOT_SUP = E_PAD // (SUP * CHUNK)
SUP_C0 = 18
SUP_C1 = TOT_SUP // NS - SUP_C0

N_PAD = 10240
ROWS_PER_SUB = N_PAD // NS
ZCHUNK = 128

NBUF = 2


def _make_spmm(D):
    mesh = plsc.VectorSubcoreMesh(core_axis_name="c", subcore_axis_name="s")

    cp = pltpu.CompilerParams()
    if "needs_layout_passes" in pltpu.CompilerParams.__dataclass_fields__:
        cp = dataclasses.replace(cp, needs_layout_passes=False)

    @functools.partial(
        pl.kernel,
        compiler_params=cp,
        out_type=jax.ShapeDtypeStruct((NC, N_PAD, D), jnp.float32),
        mesh=mesh,
        scratch_types=[
            pltpu.VMEM((SUP, CHUNK), jnp.int32),
            pltpu.VMEM((SUP, CHUNK), jnp.int32),
            pltpu.VMEM((SUP, CHUNK), jnp.int32),
            pltpu.VMEM((SUP, CHUNK), jnp.int32),
            pltpu.VMEM((SUP * CHUNK,), jnp.float32),
            pltpu.VMEM((SUP * CHUNK,), jnp.float32),
            pltpu.VMEM((CHUNK, D), jnp.float32),
            pltpu.VMEM((CHUNK, D), jnp.float32),
            pltpu.VMEM_SHARED((N_PAD, D), jnp.float32),
            pltpu.SemaphoreType.DMA,
            pltpu.SemaphoreType.DMA,
            pltpu.SemaphoreType.DMA,
            pltpu.SemaphoreType.DMA,
        ],
    )
    def spmm(h_hbm, src_hbm, dst_hbm, w_hbm, out_hbm,
             src0, src1, dst0, dst1, w0, w1, r0, r1,
             acc_sh, sem_i0, sem_i1, g0, g1):
        cid = lax.axis_index("c")
        sid = lax.axis_index("s")
        nsup = jnp.where(cid == 0, SUP_C0, SUP_C1)
        base = jnp.where(cid == 0, sid * SUP_C0, NS * SUP_C0 + sid * SUP_C1)
        rows = (r0, r1)
        gsem = (g0, g1)

        zeros16 = jnp.zeros((LANES,), jnp.float32)

        @pl.loop(0, ZCHUNK)
        def _(r):
            for c in range(D // LANES):
                r0.at[r, pl.ds(c * LANES, LANES)][...] = zeros16

        for k in range(ROWS_PER_SUB // ZCHUNK):
            pltpu.sync_copy(
                r0, acc_sh.at[pl.ds(sid * ROWS_PER_SUB + k * ZCHUNK, ZCHUNK)])
        plsc.subcore_barrier()

        def fetch_idx(sup, sb, db, wb, sem):
            g = (base + sup) * SUP
            pltpu.async_copy(src_hbm.at[pl.ds(g, SUP)], sb, sem)
            pltpu.async_copy(dst_hbm.at[pl.ds(g, SUP)], db, sem)
            pltpu.async_copy(w_hbm.at[pl.ds(g * CHUNK, SUP * CHUNK)], wb, sem)

        def wait_idx(sb, db, wb, sem):
            pltpu.make_async_copy(src_hbm.at[pl.ds(0, SUP)], sb, sem).wait()
            pltpu.make_async_copy(dst_hbm.at[pl.ds(0, SUP)], db, sem).wait()
            pltpu.make_async_copy(
                w_hbm.at[pl.ds(0, SUP * CHUNK)], wb, sem).wait()

        def do_super(sb, db, wb):
            for j in range(NBUF - 1):
                pltpu.async_copy(h_hbm.at[sb.at[j]], rows[j], gsem[j])
            for j in range(SUP):
                p = j % NBUF
                rb = rows[p]
                pltpu.make_async_copy(h_hbm.at[sb.at[0]], rb, gsem[p]).wait()
                if j + NBUF - 1 < SUP:
                    q = (j + NBUF - 1) % NBUF
                    pltpu.async_copy(h_hbm.at[sb.at[j + NBUF - 1]],
                                     rows[q], gsem[q])

                @plsc.parallel_loop(0, CHUNK, unroll=4)
                def _(e):
                    widx = jnp.full((LANES,), j * CHUNK + e, dtype=jnp.int32)
                    wv = plsc.load_gather(wb, [widx])
                    for c in range(D // LANES):
                        sl = pl.ds(c * LANES, LANES)
                        rb.at[e, sl][...] = rb.at[e, sl][...] * wv

                pltpu.sync_copy(rb, acc_sh.at[db.at[j]], add=True)

        fetch_idx(0, src0, dst0, w0, sem_i0)
        nsup_even = (nsup // 2) * 2

        @pl.loop(0, nsup_even, step=2)
        def _(sup):
            wait_idx(src0, dst0, w0, sem_i0)
            fetch_idx(sup + 1, src1, dst1, w1, sem_i1)
            do_super(src0, dst0, w0)
            wait_idx(src1, dst1, w1, sem_i1)

            @pl.when(sup + 2 < nsup)
            def _():
                fetch_idx(sup + 2, src0, dst0, w0, sem_i0)

            do_super(src1, dst1, w1)

        @pl.when(nsup_even < nsup)
        def _():
            wait_idx(src0, dst0, w0, sem_i0)
            do_super(src0, dst0, w0)

        plsc.subcore_barrier()
        for k in range(ROWS_PER_SUB // ZCHUNK):
            row0 = sid * ROWS_PER_SUB + k * ZCHUNK
            pltpu.sync_copy(acc_sh.at[pl.ds(row0, ZCHUNK)],
                            out_hbm.at[cid, pl.ds(row0, ZCHUNK)])

    return spmm


_spmm = _make_spmm(NHID)

_ROWS = 1000


def _mm1(x, W1):
    def body(x_ref, w_ref, o_ref):
        o_ref[...] = jnp.dot(x_ref[...], w_ref[...],
                             preferred_element_type=jnp.float32)

    return pl.pallas_call(
        body,
        grid=(N // _ROWS,),
        in_specs=[pl.BlockSpec((_ROWS, NFEAT), lambda i: (i, 0)),
                  pl.BlockSpec((NFEAT, NHID), lambda i: (0, 0))],
        out_specs=pl.BlockSpec((_ROWS, NHID), lambda i: (i, 0)),
        out_shape=jax.ShapeDtypeStruct((N, NHID), jnp.float32),
    )(x, W1)


def _mid(p, b1, W2):
    def body(p_ref, b_ref, w_ref, o_ref):
        h = p_ref[0] + p_ref[1] + b_ref[...]
        h = jnp.maximum(h, 0.0)
        o_ref[...] = jnp.dot(h, w_ref[...], preferred_element_type=jnp.float32)

    return pl.pallas_call(
        body,
        grid=(N_PAD // _ROWS,),
        in_specs=[pl.BlockSpec((NC, _ROWS, NHID), lambda i: (0, i, 0)),
                  pl.BlockSpec((1, NHID), lambda i: (0, 0)),
                  pl.BlockSpec((NHID, NHID), lambda i: (0, 0))],
        out_specs=pl.BlockSpec((_ROWS, NHID), lambda i: (i, 0)),
        out_shape=jax.ShapeDtypeStruct((N_PAD, NHID), jnp.float32),
    )(p, b1, W2)


def _logsoftmax_out(q, b2):
    def body(q_ref, b_ref, o_ref):
        z = q_ref[0, :, :NCLASS] + q_ref[1, :, :NCLASS] + b_ref[...]
        m = jnp.max(z, axis=1, keepdims=True)
        s = jnp.sum(jnp.exp(z - m), axis=1, keepdims=True)
        o_ref[...] = z - m - jnp.log(s)

    return pl.pallas_call(
        body,
        grid=(N // _ROWS,),
        in_specs=[pl.BlockSpec((NC, _ROWS, NHID), lambda i: (0, i, 0)),
                  pl.BlockSpec((1, NCLASS), lambda i: (0, 0))],
        out_specs=pl.BlockSpec((_ROWS, NCLASS), lambda i: (i, 0)),
        out_shape=jax.ShapeDtypeStruct((N, NCLASS), jnp.float32),
    )(q, b2)


def kernel(x, edge_index, edge_weight, W1, b1, W2, b2):
    dst = edge_index[0].astype(jnp.int32)
    src = edge_index[1].astype(jnp.int32)
    w = edge_weight.astype(jnp.float32)

    pad = E_PAD - E
    srcp = jnp.concatenate([src, jnp.zeros((pad,), jnp.int32)])
    dstp = jnp.concatenate([dst, jnp.zeros((pad,), jnp.int32)])
    wp = jnp.concatenate([w, jnp.zeros((pad,), jnp.float32)])
    srcp = srcp.reshape(E_PAD // CHUNK, CHUNK)
    dstp = dstp.reshape(E_PAD // CHUNK, CHUNK)

    W2p = jnp.pad(W2, ((0, 0), (0, NHID - NCLASS)))

    support1 = _mm1(x, W1)
    p = _spmm(support1, srcp, dstp, wp)
    h2 = _mid(p, b1.reshape(1, NHID), W2p)
    q = _spmm(h2, srcp, dstp, wp)
    return _logsoftmax_out(q, b2.reshape(1, NCLASS))
